# Initial kernel scaffold; baseline (speedup 1.0000x reference)
import jax
import jax.numpy as jnp
from jax.experimental import pallas as pl


def kernel(x_user, x_item, edge_index_user_item, edge_index_item_user, W_user, b_user, W_item, b_item):
    raise NotImplementedError("write your pallas kernel here")


# speedup vs baseline: 9.5060x; 9.5060x over previous
import functools

import jax
import jax.numpy as jnp
from jax import lax
from jax.experimental import pallas as pl
from jax.experimental.pallas import tpu as pltpu
from jax.experimental.pallas import tpu_sc as plsc

N_SIDE = 5000
D = 128
E_SIDE = 160000
NC, NS, L = 2, 16, 16
NP = 5120
RPT = NP // NS
K = 128
EP = 163840
CH = EP // (NS * K)

_f32 = jnp.float32
_i32 = jnp.int32

_mesh = plsc.VectorSubcoreMesh(core_axis_name="c", subcore_axis_name="s",
                               num_cores=NC, num_subcores=NS)


@functools.partial(
    pl.kernel,
    out_type=[jax.ShapeDtypeStruct((NP, D), _f32),
              jax.ShapeDtypeStruct((NP, D), _f32)],
    mesh=_mesh,
    scratch_types=[
        pltpu.VMEM_SHARED((NP, D), _f32),
        pltpu.VMEM((CH, K), _i32),
        pltpu.VMEM((K, D), _f32),
    ],
)
def _deg_sc(dstA, dstB, ones_hbm, zero16_hbm, outA, outB, acc, dstv, onesv):
    c = lax.axis_index("c")
    s = lax.axis_index("s")

    def side(dstm, out):
        pltpu.sync_copy(zero16_hbm, acc.at[pl.ds(s * RPT, RPT)])
        pltpu.sync_copy(dstm.at[s], dstv)
        pltpu.sync_copy(ones_hbm, onesv)
        plsc.subcore_barrier()

        def body(j, carry):
            pltpu.sync_copy(onesv, acc.at[dstv.at[j]], add=True)
            return carry

        lax.fori_loop(0, CH, body, 0)
        plsc.subcore_barrier()
        pltpu.sync_copy(acc.at[pl.ds(s * RPT, RPT)], out.at[pl.ds(s * RPT, RPT)])

    @pl.when(c == 0)
    def _():
        side(dstA, outA)

    @pl.when(c == 1)
    def _():
        side(dstB, outB)


@functools.partial(
    pl.kernel,
    out_type=[jax.ShapeDtypeStruct((NP, D), _f32),
              jax.ShapeDtypeStruct((NP, D), _f32)],
    mesh=_mesh,
    scratch_types=[
        pltpu.VMEM_SHARED((NP, D), _f32),
        pltpu.VMEM((CH, K), _i32),
        pltpu.VMEM((CH, K), _i32),
        pltpu.VMEM((K, D), _f32),
        pltpu.SemaphoreType.DMA,
    ],
)
def _layer_sc(tabA, srcA, dstA, tabB, srcB, dstB, zero_hbm,
              outA, outB, acc, srcv, dstv, rows, sem):
    c = lax.axis_index("c")
    s = lax.axis_index("s")

    def side(tab, srcm, dstm, out):
        pltpu.sync_copy(zero_hbm, acc.at[pl.ds(s * RPT, RPT)])
        pltpu.sync_copy(srcm.at[s], srcv)
        pltpu.sync_copy(dstm.at[s], dstv)
        plsc.subcore_barrier()

        def body(j, carry):
            pltpu.async_copy(tab.at[srcv.at[j]], rows, sem).wait()
            pltpu.sync_copy(rows, acc.at[dstv.at[j]], add=True)
            return carry

        lax.fori_loop(0, CH, body, 0)
        plsc.subcore_barrier()
        pltpu.sync_copy(acc.at[pl.ds(s * RPT, RPT)], out.at[pl.ds(s * RPT, RPT)])

    @pl.when(c == 0)
    def _():
        side(tabA, srcA, dstA, outA)

    @pl.when(c == 1)
    def _():
        side(tabB, srcB, dstB, outB)


def _prep_tc(xu, wu, bu, xi, wi, bi, dgu, dgi,
             zu, zi, u0u, u0i, dbu, dbi):
    dn = (((1,), (1,)), ((), ()))
    zuv = lax.dot_general(xu[...], wu[...], dn, preferred_element_type=_f32)
    zuv = zuv + bu[...]
    ziv = lax.dot_general(xi[...], wi[...], dn, preferred_element_type=_f32)
    ziv = ziv + bi[...]
    du = dgu[...]
    di = dgi[...]
    disu = jnp.where(du > 0, lax.rsqrt(du), 0.0)
    disi = jnp.where(di > 0, lax.rsqrt(di), 0.0)
    zu[...] = zuv
    zi[...] = ziv
    u0u[...] = zuv * disu
    u0i[...] = ziv * disi
    dbu[...] = disu
    dbi[...] = disi


def _mid_tc(dbu, t1u, dbi, t1i, u1u, u1i):
    u1u[...] = dbu[...] * dbu[...] * t1u[...]
    u1i[...] = dbi[...] * dbi[...] * t1i[...]


def _final_tc(zu, dbu, t1u, t2u, zi, dbi, t1i, t2i, fu, fi):
    third = jnp.float32(1.0 / 3.0)
    fu[...] = (zu[...] + dbu[...] * (t1u[...] + t2u[...])) * third
    fi[...] = (zi[...] + dbi[...] * (t1i[...] + t2i[...])) * third


_sds = jax.ShapeDtypeStruct
_prep_call = pl.pallas_call(
    _prep_tc, out_shape=[_sds((NP, D), _f32)] * 6)
_mid_call = pl.pallas_call(
    _mid_tc, out_shape=[_sds((NP, D), _f32)] * 2)
_final_call = pl.pallas_call(
    _final_tc, out_shape=[_sds((NP, D), _f32)] * 2)


def kernel(x_user, x_item, edge_index_user_item, edge_index_item_user,
           W_user, b_user, W_item, b_item):
    padn = NP - N_SIDE
    pade = EP - E_SIDE
    xu = jnp.pad(x_user, ((0, padn), (0, 0)))
    xi = jnp.pad(x_item, ((0, padn), (0, 0)))

    def edges3(idx, fill):
        idx = jnp.concatenate([idx, jnp.full((pade,), fill, _i32)])
        return idx.reshape(NS, CH, K)

    uis = edges3(edge_index_user_item[0], 0)
    uid = edges3(edge_index_user_item[1], NP - 1)
    ius = edges3(edge_index_item_user[0], 0)
    iud = edges3(edge_index_item_user[1], NP - 1)

    onesK = jnp.ones((K, D), _f32)
    zeroD = jnp.zeros((RPT, D), _f32)

    deg_item, deg_user = _deg_sc(uid, iud, onesK, zeroD)

    zu, zi, u0u, u0i, dbu, dbi = _prep_call(
        xu, W_user, b_user.reshape(1, D), xi, W_item, b_item.reshape(1, D),
        deg_user, deg_item)

    t1i, t1u = _layer_sc(u0u, uis, uid, u0i, ius, iud, zeroD)
    u1u, u1i = _mid_call(dbu, t1u, dbi, t1i)
    t2i, t2u = _layer_sc(u1u, uis, uid, u1i, ius, iud, zeroD)
    fu, fi = _final_call(zu, dbu, t1u, t2u, zi, dbi, t1i, t2i)
    return (fu[:N_SIDE], fi[:N_SIDE])


# speedup vs baseline: 10.5805x; 1.1130x over previous
import functools

import jax
import jax.numpy as jnp
from jax import lax
from jax.experimental import pallas as pl
from jax.experimental.pallas import tpu as pltpu
from jax.experimental.pallas import tpu_sc as plsc

N_SIDE = 5000
D = 128
E_SIDE = 160000
NC, NS, L = 2, 16, 16
NP = 5120
RPT = NP // NS
K = 128
EP = 163840
CH = EP // (NS * K)

_f32 = jnp.float32
_i32 = jnp.int32

_mesh = plsc.VectorSubcoreMesh(core_axis_name="c", subcore_axis_name="s",
                               num_cores=NC, num_subcores=NS)


@functools.partial(
    pl.kernel,
    out_type=[jax.ShapeDtypeStruct((NP, D), _f32),
              jax.ShapeDtypeStruct((NP, D), _f32)],
    mesh=_mesh,
    scratch_types=[
        pltpu.VMEM_SHARED((NP, D), _f32),
        pltpu.VMEM((CH, K), _i32),
        pltpu.VMEM((K, D), _f32),
    ],
)
def _deg_sc(dstA, dstB, ones_hbm, zero16_hbm, outA, outB, acc, dstv, onesv):
    c = lax.axis_index("c")
    s = lax.axis_index("s")

    def side(dstm, out):
        pltpu.sync_copy(zero16_hbm, acc.at[pl.ds(s * RPT, RPT)])
        pltpu.sync_copy(dstm.at[s], dstv)
        pltpu.sync_copy(ones_hbm, onesv)
        plsc.subcore_barrier()

        def body(j, carry):
            pltpu.sync_copy(onesv, acc.at[dstv.at[j]], add=True)
            return carry

        lax.fori_loop(0, CH, body, 0)
        plsc.subcore_barrier()
        pltpu.sync_copy(acc.at[pl.ds(s * RPT, RPT)], out.at[pl.ds(s * RPT, RPT)])

    @pl.when(c == 0)
    def _():
        side(dstA, outA)

    @pl.when(c == 1)
    def _():
        side(dstB, outB)


@functools.partial(
    pl.kernel,
    out_type=[jax.ShapeDtypeStruct((NP, D), _f32),
              jax.ShapeDtypeStruct((NP, D), _f32)],
    mesh=_mesh,
    scratch_types=[
        pltpu.VMEM_SHARED((NP, D), _f32),
        pltpu.VMEM((CH, K), _i32),
        pltpu.VMEM((CH, K), _i32),
        pltpu.VMEM((K, D), _f32),
        pltpu.VMEM((K, D), _f32),
        pltpu.SemaphoreType.DMA,
    ],
)
def _layer_sc(tabA, srcA, dstA, tabB, srcB, dstB, zero_hbm,
              outA, outB, acc, srcv, dstv, rows0, rows1, sem):
    c = lax.axis_index("c")
    s = lax.axis_index("s")

    def side(tab, srcm, dstm, out):
        pltpu.sync_copy(zero_hbm, acc.at[pl.ds(s * RPT, RPT)])
        pltpu.sync_copy(srcm.at[s], srcv)
        pltpu.sync_copy(dstm.at[s], dstv)
        plsc.subcore_barrier()

        pltpu.async_copy(tab.at[srcv.at[0]], rows0, sem)

        def body(i, carry):
            j = 2 * i
            pltpu.make_async_copy(tab.at[srcv.at[j]], rows0, sem).wait()
            pltpu.async_copy(tab.at[srcv.at[j + 1]], rows1, sem)
            pltpu.sync_copy(rows0, acc.at[dstv.at[j]], add=True)
            pltpu.make_async_copy(tab.at[srcv.at[j]], rows1, sem).wait()

            @pl.when(j + 2 < CH)
            def _():
                pltpu.async_copy(tab.at[srcv.at[j + 2]], rows0, sem)

            pltpu.sync_copy(rows1, acc.at[dstv.at[j + 1]], add=True)
            return carry

        lax.fori_loop(0, CH // 2, body, 0)
        plsc.subcore_barrier()
        pltpu.sync_copy(acc.at[pl.ds(s * RPT, RPT)], out.at[pl.ds(s * RPT, RPT)])

    @pl.when(c == 0)
    def _():
        side(tabA, srcA, dstA, outA)

    @pl.when(c == 1)
    def _():
        side(tabB, srcB, dstB, outB)


def _prep_tc(xu, wu, bu, xi, wi, bi, dgu, dgi,
             zu, zi, u0u, u0i, dbu, dbi):
    dn = (((1,), (1,)), ((), ()))
    zuv = lax.dot_general(xu[...], wu[...], dn, preferred_element_type=_f32)
    zuv = zuv + bu[...]
    ziv = lax.dot_general(xi[...], wi[...], dn, preferred_element_type=_f32)
    ziv = ziv + bi[...]
    du = dgu[...]
    di = dgi[...]
    disu = jnp.where(du > 0, lax.rsqrt(du), 0.0)
    disi = jnp.where(di > 0, lax.rsqrt(di), 0.0)
    zu[...] = zuv
    zi[...] = ziv
    u0u[...] = zuv * disu
    u0i[...] = ziv * disi
    dbu[...] = disu
    dbi[...] = disi


def _mid_tc(dbu, t1u, dbi, t1i, u1u, u1i):
    u1u[...] = dbu[...] * dbu[...] * t1u[...]
    u1i[...] = dbi[...] * dbi[...] * t1i[...]


def _final_tc(zu, dbu, t1u, t2u, zi, dbi, t1i, t2i, fu, fi):
    third = jnp.float32(1.0 / 3.0)
    fu[...] = (zu[...] + dbu[...] * (t1u[...] + t2u[...])) * third
    fi[...] = (zi[...] + dbi[...] * (t1i[...] + t2i[...])) * third


_sds = jax.ShapeDtypeStruct
_prep_call = pl.pallas_call(
    _prep_tc, out_shape=[_sds((NP, D), _f32)] * 6)
_mid_call = pl.pallas_call(
    _mid_tc, out_shape=[_sds((NP, D), _f32)] * 2)
_final_call = pl.pallas_call(
    _final_tc, out_shape=[_sds((NP, D), _f32)] * 2)


def kernel(x_user, x_item, edge_index_user_item, edge_index_item_user,
           W_user, b_user, W_item, b_item):
    padn = NP - N_SIDE
    pade = EP - E_SIDE
    xu = jnp.pad(x_user, ((0, padn), (0, 0)))
    xi = jnp.pad(x_item, ((0, padn), (0, 0)))

    def edges3(idx, fill):
        idx = jnp.concatenate([idx, jnp.full((pade,), fill, _i32)])
        return idx.reshape(NS, CH, K)

    uis = edges3(edge_index_user_item[0], 0)
    uid = edges3(edge_index_user_item[1], NP - 1)
    ius = edges3(edge_index_item_user[0], 0)
    iud = edges3(edge_index_item_user[1], NP - 1)

    onesK = jnp.ones((K, D), _f32)
    zeroD = jnp.zeros((RPT, D), _f32)

    deg_item, deg_user = _deg_sc(uid, iud, onesK, zeroD)

    zu, zi, u0u, u0i, dbu, dbi = _prep_call(
        xu, W_user, b_user.reshape(1, D), xi, W_item, b_item.reshape(1, D),
        deg_user, deg_item)

    t1i, t1u = _layer_sc(u0u, uis, uid, u0i, ius, iud, zeroD)
    u1u, u1i = _mid_call(dbu, t1u, dbi, t1i)
    t2i, t2u = _layer_sc(u1u, uis, uid, u1i, ius, iud, zeroD)
    fu, fi = _final_call(zu, dbu, t1u, t2u, zi, dbi, t1i, t2i)
    return (fu[:N_SIDE], fi[:N_SIDE])


# speedup vs baseline: 17.6990x; 1.6728x over previous
import functools

import jax
import jax.numpy as jnp
from jax import lax
from jax.experimental import pallas as pl
from jax.experimental.pallas import tpu as pltpu
from jax.experimental.pallas import tpu_sc as plsc

N_SIDE = 5000
D = 128
E_SIDE = 160000
NC, NS, L = 2, 16, 16
NP = 5120
RPT = NP // NS
K = 128
EP = 163840
CH = EP // (NS * K)
CHH = CH // 2

_f32 = jnp.float32
_i32 = jnp.int32

_mesh = plsc.VectorSubcoreMesh(core_axis_name="c", subcore_axis_name="s",
                               num_cores=NC, num_subcores=NS)


@functools.partial(
    pl.kernel,
    out_type=jax.ShapeDtypeStruct((NC, NP, D), _f32),
    mesh=_mesh,
    scratch_types=[
        pltpu.VMEM_SHARED((NP, D), _f32),
        pltpu.VMEM((CH, K), _i32),
        pltpu.VMEM((K, D), _f32),
    ],
)
def _deg_sc(dst2, ones_hbm, zero_hbm, out, acc, dstv, onesv):
    c = lax.axis_index("c")
    s = lax.axis_index("s")
    pltpu.sync_copy(zero_hbm, acc.at[pl.ds(s * RPT, RPT)])
    pltpu.sync_copy(dst2.at[c, s], dstv)
    pltpu.sync_copy(ones_hbm, onesv)
    plsc.subcore_barrier()

    def body(j, carry):
        pltpu.sync_copy(onesv, acc.at[dstv.at[j]], add=True)
        return carry

    lax.fori_loop(0, CH, body, 0)
    plsc.subcore_barrier()
    pltpu.sync_copy(acc.at[pl.ds(s * RPT, RPT)],
                    out.at[c, pl.ds(s * RPT, RPT)])


@functools.partial(
    pl.kernel,
    out_type=jax.ShapeDtypeStruct((NC, NP, D), _f32),
    mesh=_mesh,
    scratch_types=[
        pltpu.VMEM_SHARED((NP, D), _f32),
        pltpu.VMEM_SHARED((NP, D), _f32),
        pltpu.VMEM((CHH, K), _i32),
        pltpu.VMEM((CHH, K), _i32),
        pltpu.VMEM((K, D), _f32),
        pltpu.VMEM((K, D), _f32),
        pltpu.SemaphoreType.DMA,
    ],
)
def _layer_sc(tab2, src2, dst2, zero_hbm, out,
              acc, tab_s, srcv, dstv, rows0, rows1, sem):
    c = lax.axis_index("c")
    s = lax.axis_index("s")
    pltpu.sync_copy(zero_hbm, acc.at[pl.ds(s * RPT, RPT)])
    pltpu.sync_copy(tab2.at[c, pl.ds(s * RPT, RPT)],
                    tab_s.at[pl.ds(s * RPT, RPT)])
    plsc.subcore_barrier()

    for h in range(2):
        pltpu.sync_copy(src2.at[c, s, pl.ds(h * CHH, CHH)], srcv)
        pltpu.sync_copy(dst2.at[c, s, pl.ds(h * CHH, CHH)], dstv)
        pltpu.async_copy(tab_s.at[srcv.at[0]], rows0, sem)

        def body(i, carry):
            j = 2 * i
            pltpu.make_async_copy(tab_s.at[srcv.at[j]], rows0, sem).wait()
            pltpu.async_copy(tab_s.at[srcv.at[j + 1]], rows1, sem)
            pltpu.sync_copy(rows0, acc.at[dstv.at[j]], add=True)
            pltpu.make_async_copy(tab_s.at[srcv.at[j]], rows1, sem).wait()

            @pl.when(j + 2 < CHH)
            def _():
                pltpu.async_copy(tab_s.at[srcv.at[j + 2]], rows0, sem)

            pltpu.sync_copy(rows1, acc.at[dstv.at[j + 1]], add=True)
            return carry

        lax.fori_loop(0, CHH // 2, body, 0)
    plsc.subcore_barrier()
    pltpu.sync_copy(acc.at[pl.ds(s * RPT, RPT)],
                    out.at[c, pl.ds(s * RPT, RPT)])


def _prep_tc(xu, wu, bu, xi, wi, bi, dgu, dgi,
             zu, zi, u0u, u0i, dbu, dbi):
    dn = (((1,), (1,)), ((), ()))
    zuv = lax.dot_general(xu[...], wu[...], dn, preferred_element_type=_f32)
    zuv = zuv + bu[...]
    ziv = lax.dot_general(xi[...], wi[...], dn, preferred_element_type=_f32)
    ziv = ziv + bi[...]
    du = dgu[...]
    di = dgi[...]
    disu = jnp.where(du > 0, lax.rsqrt(du), 0.0)
    disi = jnp.where(di > 0, lax.rsqrt(di), 0.0)
    zu[...] = zuv
    zi[...] = ziv
    u0u[...] = zuv * disu
    u0i[...] = ziv * disi
    dbu[...] = disu
    dbi[...] = disi


def _mid_tc(dbu, t1u, dbi, t1i, u1u, u1i):
    u1u[...] = dbu[...] * dbu[...] * t1u[...]
    u1i[...] = dbi[...] * dbi[...] * t1i[...]


def _final_tc(zu, dbu, t1u, t2u, zi, dbi, t1i, t2i, fu, fi):
    third = jnp.float32(1.0 / 3.0)
    fu[...] = (zu[...] + dbu[...] * (t1u[...] + t2u[...])) * third
    fi[...] = (zi[...] + dbi[...] * (t1i[...] + t2i[...])) * third


_sds = jax.ShapeDtypeStruct
_prep_call = pl.pallas_call(
    _prep_tc, out_shape=[_sds((NP, D), _f32)] * 6)
_mid_call = pl.pallas_call(
    _mid_tc, out_shape=[_sds((NP, D), _f32)] * 2)
_final_call = pl.pallas_call(
    _final_tc, out_shape=[_sds((NP, D), _f32)] * 2)


def kernel(x_user, x_item, edge_index_user_item, edge_index_item_user,
           W_user, b_user, W_item, b_item):
    padn = NP - N_SIDE
    pade = EP - E_SIDE
    xu = jnp.pad(x_user, ((0, padn), (0, 0)))
    xi = jnp.pad(x_item, ((0, padn), (0, 0)))

    def edges4(a, b, fa, fb):
        a = jnp.concatenate([a, jnp.full((pade,), fa, _i32)]).reshape(NS, CH, K)
        b = jnp.concatenate([b, jnp.full((pade,), fb, _i32)]).reshape(NS, CH, K)
        return jnp.stack([a, b], axis=0)

    src2 = edges4(edge_index_user_item[0], edge_index_item_user[0], 0, 0)
    dst2 = edges4(edge_index_user_item[1], edge_index_item_user[1],
                  NP - 1, NP - 1)

    onesK = jnp.ones((K, D), _f32)
    zeroD = jnp.zeros((RPT, D), _f32)

    deg2 = _deg_sc(dst2, onesK, zeroD)
    deg_item, deg_user = deg2[0], deg2[1]

    zu, zi, u0u, u0i, dbu, dbi = _prep_call(
        xu, W_user, b_user.reshape(1, D), xi, W_item, b_item.reshape(1, D),
        deg_user, deg_item)

    t1 = _layer_sc(jnp.stack([u0u, u0i]), src2, dst2, zeroD)
    t1i, t1u = t1[0], t1[1]
    u1u, u1i = _mid_call(dbu, t1u, dbi, t1i)
    t2 = _layer_sc(jnp.stack([u1u, u1i]), src2, dst2, zeroD)
    t2i, t2u = t2[0], t2[1]
    fu, fi = _final_call(zu, dbu, t1u, t2u, zi, dbi, t1i, t2i)
    return (fu[:N_SIDE], fi[:N_SIDE])
